# baseline (device time: 22505 ns/iter reference)
import jax
import jax.numpy as jnp
from jax import lax
from jax.experimental import pallas as pl
from jax.experimental.pallas import tpu as pltpu

N_DEV = 8
N_HALVES = 4


def kernel(x, w_mat):
    m_per, k = x.shape
    n = w_mat.shape[1]
    n_per = n // N_DEV
    blocks_per_half = N_DEV // N_HALVES

    def body(x_ref, w_ref, out_ref, send_buf, recv_buf,
             send_sems, recv_sems):
        me = lax.axis_index("i")

        barrier_sem = pltpu.get_barrier_semaphore()
        for d in range(1, N_DEV):
            pl.semaphore_signal(
                barrier_sem, inc=1,
                device_id=((me + d) % N_DEV,),
                device_id_type=pl.DeviceIdType.MESH,
            )
        pl.semaphore_wait(barrier_sem, N_DEV - 1)

        for h in range(N_HALVES):
            half = jnp.dot(
                x_ref[:, :],
                w_ref[:, h * (n // N_HALVES):(h + 1) * (n // N_HALVES)],
                preferred_element_type=jnp.float32,
            )
            for jj in range(blocks_per_half):
                j = h * blocks_per_half + jj
                blk = half[:, jj * n_per:(jj + 1) * n_per]
                send_buf[j, :, :] = blk.astype(jnp.bfloat16)

                @pl.when(j == me)
                def _(blk=blk):
                    out_ref[pl.ds(me * m_per, m_per), :] = blk

                @pl.when(j != me)
                def _(j=j):
                    rdma = pltpu.make_async_remote_copy(
                        src_ref=send_buf.at[j],
                        dst_ref=recv_buf.at[me],
                        send_sem=send_sems.at[j],
                        recv_sem=recv_sems.at[me],
                        device_id=(j,),
                        device_id_type=pl.DeviceIdType.MESH,
                    )
                    rdma.start()

        for d in range(1, N_DEV):
            src = (me - d) % N_DEV
            recv = pltpu.make_async_remote_copy(
                src_ref=recv_buf.at[src],
                dst_ref=recv_buf.at[src],
                send_sem=send_sems.at[src],
                recv_sem=recv_sems.at[src],
                device_id=(src,),
                device_id_type=pl.DeviceIdType.MESH,
            )
            recv.wait_recv()
            out_ref[pl.ds(src * m_per, m_per), :] = recv_buf[src].astype(
                jnp.float32
            )

        for j in range(N_DEV):
            @pl.when(j != me)
            def _(j=j):
                done = pltpu.make_async_remote_copy(
                    src_ref=send_buf.at[j],
                    dst_ref=recv_buf.at[me],
                    send_sem=send_sems.at[j],
                    recv_sem=recv_sems.at[me],
                    device_id=(j,),
                    device_id_type=pl.DeviceIdType.MESH,
                )
                done.wait_send()

    return pl.pallas_call(
        body,
        out_shape=jax.ShapeDtypeStruct((N_DEV * m_per, n_per), jnp.float32),
        in_specs=[
            pl.BlockSpec(memory_space=pltpu.VMEM),
            pl.BlockSpec(memory_space=pltpu.VMEM),
        ],
        out_specs=pl.BlockSpec(memory_space=pltpu.VMEM),
        scratch_shapes=[
            pltpu.VMEM((N_DEV, m_per, n_per), jnp.bfloat16),
            pltpu.VMEM((N_DEV, m_per, n_per), jnp.bfloat16),
            pltpu.SemaphoreType.DMA((N_DEV,)),
            pltpu.SemaphoreType.DMA((N_DEV,)),
        ],
        compiler_params=pltpu.CompilerParams(collective_id=0),
    )(x, w_mat)


# device time: 21984 ns/iter; 1.0237x vs baseline; 1.0237x over previous
import jax
import jax.numpy as jnp
from jax import lax
from jax.experimental import pallas as pl
from jax.experimental.pallas import tpu as pltpu

N_DEV = 8
N_HALVES = 2


def kernel(x, w_mat):
    m_per, k = x.shape
    n = w_mat.shape[1]
    n_per = n // N_DEV
    blocks_per_half = N_DEV // N_HALVES

    def body(x_ref, w_ref, out_ref, send_buf, recv_buf,
             send_sems, recv_sems):
        me = lax.axis_index("i")

        barrier_sem = pltpu.get_barrier_semaphore()
        for d in range(1, N_DEV):
            pl.semaphore_signal(
                barrier_sem, inc=1,
                device_id=((me + d) % N_DEV,),
                device_id_type=pl.DeviceIdType.MESH,
            )
        pl.semaphore_wait(barrier_sem, N_DEV - 1)

        for h in range(N_HALVES):
            half = jnp.dot(
                x_ref[:, :],
                w_ref[:, h * (n // N_HALVES):(h + 1) * (n // N_HALVES)],
                preferred_element_type=jnp.float32,
            )
            for jj in range(blocks_per_half):
                j = h * blocks_per_half + jj
                blk = half[:, jj * n_per:(jj + 1) * n_per]
                send_buf[j, :, :] = blk.astype(jnp.bfloat16)

                @pl.when(j == me)
                def _(blk=blk):
                    out_ref[pl.ds(me * m_per, m_per), :] = blk

                @pl.when(j != me)
                def _(j=j):
                    rdma = pltpu.make_async_remote_copy(
                        src_ref=send_buf.at[j],
                        dst_ref=recv_buf.at[me],
                        send_sem=send_sems.at[j],
                        recv_sem=recv_sems.at[me],
                        device_id=(j,),
                        device_id_type=pl.DeviceIdType.MESH,
                    )
                    rdma.start()

        for d in range(1, N_DEV):
            src = (me - d) % N_DEV
            recv = pltpu.make_async_remote_copy(
                src_ref=recv_buf.at[src],
                dst_ref=recv_buf.at[src],
                send_sem=send_sems.at[src],
                recv_sem=recv_sems.at[src],
                device_id=(src,),
                device_id_type=pl.DeviceIdType.MESH,
            )
            recv.wait_recv()
            out_ref[pl.ds(src * m_per, m_per), :] = recv_buf[src].astype(
                jnp.float32
            )

        for j in range(N_DEV):
            @pl.when(j != me)
            def _(j=j):
                done = pltpu.make_async_remote_copy(
                    src_ref=send_buf.at[j],
                    dst_ref=recv_buf.at[me],
                    send_sem=send_sems.at[j],
                    recv_sem=recv_sems.at[me],
                    device_id=(j,),
                    device_id_type=pl.DeviceIdType.MESH,
                )
                done.wait_send()

    return pl.pallas_call(
        body,
        out_shape=jax.ShapeDtypeStruct((N_DEV * m_per, n_per), jnp.float32),
        in_specs=[
            pl.BlockSpec(memory_space=pltpu.VMEM),
            pl.BlockSpec(memory_space=pltpu.VMEM),
        ],
        out_specs=pl.BlockSpec(memory_space=pltpu.VMEM),
        scratch_shapes=[
            pltpu.VMEM((N_DEV, m_per, n_per), jnp.bfloat16),
            pltpu.VMEM((N_DEV, m_per, n_per), jnp.bfloat16),
            pltpu.SemaphoreType.DMA((N_DEV,)),
            pltpu.SemaphoreType.DMA((N_DEV,)),
        ],
        compiler_params=pltpu.CompilerParams(collective_id=0),
    )(x, w_mat)
